# baseline (speedup 1.0000x reference)
import functools

import jax
import jax.numpy as jnp
from jax import lax
from jax.experimental import pallas as pl
from jax.experimental.pallas import tpu as pltpu
from jax.experimental.pallas import tpu_sc as plsc

H = 128
N_NODES = 10000
N_EDGES = 160000
NW = 32
CHUNK = 40
ACC_ROWS = 10240
NPW = ACC_ROWS // (NW // 2)
DUMMY_ROW = 10016

EHALF = N_EDGES // 2
EH = 83200
NCHUNK = 65
EPW = NCHUNK * CHUNK

EB = 2080
NB = 2000

_f32 = jnp.float32


def _ln(h, g, beta):
    mu = jnp.mean(h, axis=-1, keepdims=True)
    d = h - mu
    var = jnp.mean(d * d, axis=-1, keepdims=True)
    return d * lax.rsqrt(var + 1e-5) * g + beta


def _dot(a, b):
    return jnp.dot(a, b, preferred_element_type=_f32)


def _node_encoder_body(x_ref, noise_ref, y_ref, w0_ref, b0_ref, w1_ref, b1_ref,
                       w2_ref, b2_ref, g_ref, beta_ref, ws_ref, wr_ref,
                       h_ref, ps_ref, pr_ref, tgt_ref):
    x = x_ref[...]
    noised = x[:, 1:3] + noise_ref[...]
    node_type = x[:, 0:1].astype(jnp.int32)
    cols = lax.broadcasted_iota(jnp.int32, (x.shape[0], 16), 1)
    one_hot = jnp.where((cols - 2) == node_type, 1.0, 0.0).astype(_f32)
    one_hot = jnp.where(cols < 2, 0.0, one_hot)
    one_hot = jnp.where(cols >= 11, 0.0, one_hot)
    feats = jnp.where(cols == 0, noised[:, 0:1], one_hot)
    feats = jnp.where(cols == 1, noised[:, 1:2], feats)
    cnt = feats.shape[0]
    mean = jnp.sum(feats, axis=0, keepdims=True) / cnt
    var = jnp.maximum(jnp.sum(feats * feats, axis=0, keepdims=True) / cnt
                      - mean * mean, 0.0)
    std = jnp.maximum(jnp.sqrt(var), 1e-8)
    na = (feats - mean) / std
    h1 = jnp.maximum(_dot(na, w0_ref[...]) + b0_ref[...], 0.0)
    h2 = jnp.maximum(_dot(h1, w1_ref[...]) + b1_ref[...], 0.0)
    h3 = _dot(h2, w2_ref[...]) + b2_ref[...]
    h = _ln(h3, g_ref[...], beta_ref[...])
    h_ref[...] = h
    ps_ref[...] = _dot(h, ws_ref[...])
    pr_ref[...] = _dot(h, wr_ref[...])
    t = y_ref[...] - noised
    tm = jnp.sum(t, axis=0, keepdims=True) / cnt
    tv = jnp.maximum(jnp.sum(t * t, axis=0, keepdims=True) / cnt - tm * tm, 0.0)
    ts = jnp.maximum(jnp.sqrt(tv), 1e-8)
    tgt_ref[...] = (t - tm) / ts


def _mlp_body(x_ref, w0_ref, b0_ref, w1_ref, b1_ref, w2_ref, b2_ref,
              g_ref, beta_ref, out_ref):
    h1 = jnp.maximum(_dot(x_ref[...], w0_ref[...]) + b0_ref[...], 0.0)
    h2 = jnp.maximum(_dot(h1, w1_ref[...]) + b1_ref[...], 0.0)
    h3 = _dot(h2, w2_ref[...]) + b2_ref[...]
    out_ref[...] = _ln(h3, g_ref[...], beta_ref[...])


def _edge_mlp_body(e_ref, gs_ref, gr_ref, w0_ref, b0_ref, w1_ref, b1_ref,
                   w2_ref, b2_ref, g_ref, beta_ref, enew_ref, eout_ref):
    e = e_ref[...]
    u = _dot(e, w0_ref[...]) + gs_ref[...] + gr_ref[...] + b0_ref[...]
    h1 = jnp.maximum(u, 0.0)
    h2 = jnp.maximum(_dot(h1, w1_ref[...]) + b1_ref[...], 0.0)
    h3 = _dot(h2, w2_ref[...]) + b2_ref[...]
    en = _ln(h3, g_ref[...], beta_ref[...])
    enew_ref[...] = en
    eout_ref[...] = e + en


def _node_mlp_body(h_ref, a0_ref, a1_ref, a2_ref, a3_ref, w0h_ref, w0a_ref,
                   b0_ref, w1_ref, b1_ref, w2_ref, b2_ref, g_ref, beta_ref,
                   ws_ref, wr_ref, hout_ref, ps_ref, pr_ref):
    h = h_ref[...]
    agg = (a0_ref[0] + a1_ref[0]) + (a2_ref[0] + a3_ref[0])
    u = _dot(h, w0h_ref[...]) + _dot(agg, w0a_ref[...]) + b0_ref[...]
    h1 = jnp.maximum(u, 0.0)
    h2 = jnp.maximum(_dot(h1, w1_ref[...]) + b1_ref[...], 0.0)
    h3 = _dot(h2, w2_ref[...]) + b2_ref[...]
    hn = _ln(h3, g_ref[...], beta_ref[...])
    hout = h + hn
    hout_ref[...] = hout
    if ps_ref is not None:
        ps_ref[...] = _dot(hout, ws_ref[...])
        pr_ref[...] = _dot(hout, wr_ref[...])


def _node_mlp_last_body(h_ref, a0_ref, a1_ref, a2_ref, a3_ref, w0h_ref,
                        w0a_ref, b0_ref, w1_ref, b1_ref, w2_ref, b2_ref,
                        g_ref, beta_ref, hout_ref):
    _node_mlp_body(h_ref, a0_ref, a1_ref, a2_ref, a3_ref, w0h_ref, w0a_ref,
                   b0_ref, w1_ref, b1_ref, w2_ref, b2_ref, g_ref, beta_ref,
                   None, None, hout_ref, None, None)


def _decoder_body(h_ref, w0_ref, b0_ref, w1_ref, b1_ref, w2_ref, b2_ref,
                  out_ref):
    h1 = jnp.maximum(_dot(h_ref[...], w0_ref[...]) + b0_ref[...], 0.0)
    h2 = jnp.maximum(_dot(h1, w1_ref[...]) + b1_ref[...], 0.0)
    out_ref[...] = _dot(h2, w2_ref[...]) + b2_ref[...]


def _full(shape):
    return pl.BlockSpec(shape, lambda *_: tuple(0 for _ in shape))


def _node_encoder(x, noise, y, p, ws, wr):
    n = x.shape[0]
    w0p = jnp.zeros((16, H), _f32).at[:p['W0'].shape[0]].set(p['W0'])
    return pl.pallas_call(
        _node_encoder_body,
        out_shape=[jax.ShapeDtypeStruct((n, H), _f32),
                   jax.ShapeDtypeStruct((n, H), _f32),
                   jax.ShapeDtypeStruct((n, H), _f32),
                   jax.ShapeDtypeStruct((n, 2), _f32)],
    )(x, noise, y, w0p, p['b0'], p['W1'], p['b1'], p['W2'], p['b2'],
      p['g'], p['beta'], ws, wr)


def _edge_encoder(edge_attr, p):
    m, k = edge_attr.shape
    grid = (m // EB,)
    w = [_full((k, H)), _full((H,)), _full((H, H)), _full((H,)),
         _full((H, H)), _full((H,)), _full((H,)), _full((H,))]
    return pl.pallas_call(
        _mlp_body,
        grid=grid,
        in_specs=[pl.BlockSpec((EB, k), lambda i: (i, 0))] + w,
        out_specs=pl.BlockSpec((EB, H), lambda i: (i, 0)),
        out_shape=jax.ShapeDtypeStruct((m, H), _f32),
    )(edge_attr, p['W0'], p['b0'], p['W1'], p['b1'], p['W2'], p['b2'],
      p['g'], p['beta'])


def _edge_mlp(e, gs, gr, p):
    m = e.shape[0]
    grid = (m // EB,)
    blk = pl.BlockSpec((EB, H), lambda i: (i, 0))
    w = [_full((H, H)), _full((H,)), _full((H, H)), _full((H,)),
         _full((H, H)), _full((H,)), _full((H,)), _full((H,))]
    w0e = p['W0'][:H]
    return pl.pallas_call(
        _edge_mlp_body,
        grid=grid,
        in_specs=[blk, blk, blk] + w,
        out_specs=[blk, blk],
        out_shape=[jax.ShapeDtypeStruct((m, H), _f32),
                   jax.ShapeDtypeStruct((m, H), _f32)],
    )(e, gs, gr, w0e, p['b0'], p['W1'], p['b1'], p['W2'], p['b2'],
      p['g'], p['beta'])


def _node_mlp(h, aggA, aggB, p, ws, wr):
    n = h.shape[0]
    grid = (n // NB,)
    blk = pl.BlockSpec((NB, H), lambda i: (i, 0))
    ablk0 = pl.BlockSpec((1, NB, H), lambda i: (0, i, 0))
    ablk1 = pl.BlockSpec((1, NB, H), lambda i: (1, i, 0))
    w = [_full((H, H)), _full((H, H)), _full((H,)), _full((H, H)),
         _full((H,)), _full((H, H)), _full((H,)), _full((H,)), _full((H,))]
    w0h = p['W0'][:H]
    w0a = p['W0'][H:]
    if ws is None:
        return pl.pallas_call(
            _node_mlp_last_body,
            grid=grid,
            in_specs=[blk, ablk0, ablk1, ablk0, ablk1] + w,
            out_specs=blk,
            out_shape=jax.ShapeDtypeStruct((n, H), _f32),
        )(h, aggA, aggA, aggB, aggB, w0h, w0a, p['b0'], p['W1'], p['b1'],
          p['W2'], p['b2'], p['g'], p['beta'])
    return pl.pallas_call(
        _node_mlp_body,
        grid=grid,
        in_specs=[blk, ablk0, ablk1, ablk0, ablk1] + w
        + [_full((H, H)), _full((H, H))],
        out_specs=[blk, blk, blk],
        out_shape=[jax.ShapeDtypeStruct((n, H), _f32),
                   jax.ShapeDtypeStruct((n, H), _f32),
                   jax.ShapeDtypeStruct((n, H), _f32)],
    )(h, aggA, aggA, aggB, aggB, w0h, w0a, p['b0'], p['W1'], p['b1'],
      p['W2'], p['b2'], p['g'], p['beta'], ws, wr)


def _decoder(h, p):
    n = h.shape[0]
    grid = (n // NB,)
    w = [_full((H, H)), _full((H,)), _full((H, H)), _full((H,)),
         _full((H, 2)), _full((2,))]
    return pl.pallas_call(
        _decoder_body,
        grid=grid,
        in_specs=[pl.BlockSpec((NB, H), lambda i: (i, 0))] + w,
        out_specs=pl.BlockSpec((NB, 2), lambda i: (i, 0)),
        out_shape=jax.ShapeDtypeStruct((n, 2), _f32),
    )(h, p['W0'], p['b0'], p['W1'], p['b1'], p['W2'], p['b2'])


def _sc_gather_body(ps_hbm, pr_hbm, s3_hbm, r3_hbm, gs_hbm, gr_hbm,
                    sidx, ridx, bufa0, bufb0, bufa1, bufb1, sema0, semb0,
                    sema1, semb1):
    c = lax.axis_index("c")
    s = lax.axis_index("s")
    wid = s * 2 + c
    base = wid * EPW
    pltpu.sync_copy(s3_hbm.at[wid], sidx)
    pltpu.sync_copy(r3_hbm.at[wid], ridx)

    def fire(j, ba, bb, sa, sb):
        pltpu.async_copy(ps_hbm.at[sidx.at[j]], ba, sa)
        pltpu.async_copy(pr_hbm.at[ridx.at[j]], bb, sb)

    def drain(j, ba, bb, sa, sb):
        pltpu.make_async_copy(ps_hbm.at[sidx.at[0]], ba, sa).wait()
        pltpu.make_async_copy(pr_hbm.at[ridx.at[0]], bb, sb).wait()
        pltpu.sync_copy(ba, gs_hbm.at[pl.ds(base + j * CHUNK, CHUNK)])
        pltpu.sync_copy(bb, gr_hbm.at[pl.ds(base + j * CHUNK, CHUNK)])

    fire(0, bufa0, bufb0, sema0, semb0)

    def body(i, carry):
        j = 2 * i
        fire(j + 1, bufa1, bufb1, sema1, semb1)
        drain(j, bufa0, bufb0, sema0, semb0)
        fire(j + 2, bufa0, bufb0, sema0, semb0)
        drain(j + 1, bufa1, bufb1, sema1, semb1)
        return carry

    lax.fori_loop(0, (NCHUNK - 1) // 2, body, 0, unroll=False)
    drain(NCHUNK - 1, bufa0, bufb0, sema0, semb0)


@functools.cache
def _gather_kernel():
    return pl.kernel(
        _sc_gather_body,
        out_type=[jax.ShapeDtypeStruct((EH, H), _f32),
                  jax.ShapeDtypeStruct((EH, H), _f32)],
        mesh=plsc.VectorSubcoreMesh(core_axis_name="c", subcore_axis_name="s"),
        scratch_types=[
            pltpu.VMEM((NCHUNK, CHUNK), jnp.int32),
            pltpu.VMEM((NCHUNK, CHUNK), jnp.int32),
            pltpu.VMEM((CHUNK, H), _f32),
            pltpu.VMEM((CHUNK, H), _f32),
            pltpu.VMEM((CHUNK, H), _f32),
            pltpu.VMEM((CHUNK, H), _f32),
            pltpu.SemaphoreType.DMA,
            pltpu.SemaphoreType.DMA,
            pltpu.SemaphoreType.DMA,
            pltpu.SemaphoreType.DMA,
        ],
    )


def _sc_gather(ps, pr, s3, r3):
    return _gather_kernel()(ps, pr, s3, r3)


def _sc_scatter_body(enew_hbm, r3_hbm, out_hbm, ridx, rbuf0, rbuf1, zbuf,
                     acc, sem0, sem1):
    c = lax.axis_index("c")
    s = lax.axis_index("s")
    wid = s * 2 + c
    base = wid * EPW

    zv = jnp.zeros((16,), _f32)

    def zb(i, carry):
        r = i // 8
        col = lax.rem(i, 8) * 16
        zbuf[r, pl.ds(col, 16)] = zv
        return carry

    lax.fori_loop(0, 128 * 8, zb, 0, unroll=False)
    for k in range(5):
        pltpu.sync_copy(zbuf, acc.at[pl.ds(s * NPW + k * 128, 128)])
    plsc.subcore_barrier()

    pltpu.sync_copy(r3_hbm.at[wid], ridx)

    def fire(j, rb, sem):
        pltpu.async_copy(enew_hbm.at[pl.ds(base + j * CHUNK, CHUNK)], rb, sem)

    def drain(j, rb, sem):
        pltpu.make_async_copy(
            enew_hbm.at[pl.ds(base, CHUNK)], rb, sem).wait()
        pltpu.sync_copy(rb, acc.at[ridx.at[j]], add=True)

    fire(0, rbuf0, sem0)

    def body(i, carry):
        j = 2 * i
        fire(j + 1, rbuf1, sem1)
        drain(j, rbuf0, sem0)
        fire(j + 2, rbuf0, sem0)
        drain(j + 1, rbuf1, sem1)
        return carry

    lax.fori_loop(0, (NCHUNK - 1) // 2, body, 0, unroll=False)
    drain(NCHUNK - 1, rbuf0, sem0)
    plsc.subcore_barrier()
    for k in range(5):
        sl = pl.ds(s * NPW + k * 128, 128)
        pltpu.sync_copy(acc.at[sl], out_hbm.at[c].at[sl])


@functools.cache
def _scatter_kernel():
    return pl.kernel(
        _sc_scatter_body,
        out_type=jax.ShapeDtypeStruct((2, ACC_ROWS, H), _f32),
        mesh=plsc.VectorSubcoreMesh(core_axis_name="c", subcore_axis_name="s"),
        scratch_types=[
            pltpu.VMEM((NCHUNK, CHUNK), jnp.int32),
            pltpu.VMEM((CHUNK, H), _f32),
            pltpu.VMEM((CHUNK, H), _f32),
            pltpu.VMEM((128, H), _f32),
            pltpu.VMEM_SHARED((ACC_ROWS, H), _f32),
            pltpu.SemaphoreType.DMA,
            pltpu.SemaphoreType.DMA,
        ],
    )


def _sc_scatter(enew, r3):
    return _scatter_kernel()(enew, r3)


def kernel(x, y, edge_attr, velocity_sequence_noise, params, edge_index):
    senders = edge_index[0]
    receivers = edge_index[1]
    npad = EH - EHALF

    def halves(idx, pad_val):
        a = jnp.concatenate(
            [idx[:EHALF], jnp.full((npad,), pad_val, jnp.int32)])
        b = jnp.concatenate(
            [idx[EHALF:], jnp.full((npad,), pad_val, jnp.int32)])
        return (a.reshape(NW, NCHUNK, CHUNK), b.reshape(NW, NCHUNK, CHUNK))

    s3 = halves(senders, 0)
    r3g = halves(receivers, 0)
    r3s = halves(receivers, DUMMY_ROW)

    attr_pad = jnp.zeros((npad, edge_attr.shape[1]), _f32)
    attr_h = (jnp.concatenate([edge_attr[:EHALF], attr_pad]),
              jnp.concatenate([edge_attr[EHALF:], attr_pad]))

    gn = params['gn']
    w0 = gn[0]['edge']['W0']
    h, ps, pr, tgt = _node_encoder(x, velocity_sequence_noise, y,
                                   params['nb_enc'], w0[H:2 * H], w0[2 * H:])
    e = [_edge_encoder(attr_h[0], params['eb_enc']),
         _edge_encoder(attr_h[1], params['eb_enc'])]

    for r in range(15):
        wts = gn[r]['edge']
        agg = [None, None]
        gsA, grA = _sc_gather(ps, pr, s3[0], r3g[0])
        gsB, grB = _sc_gather(ps, pr, s3[1], r3g[1])
        e_newA, e[0] = _edge_mlp(e[0], gsA, grA, wts)
        agg[0] = _sc_scatter(e_newA, r3s[0])
        e_newB, e[1] = _edge_mlp(e[1], gsB, grB, wts)
        agg[1] = _sc_scatter(e_newB, r3s[1])
        if r < 14:
            w0n = gn[r + 1]['edge']['W0']
            h, ps, pr = _node_mlp(h, agg[0], agg[1], gn[r]['node'],
                                  w0n[H:2 * H], w0n[2 * H:])
        else:
            h = _node_mlp(h, agg[0], agg[1], gn[r]['node'], None, None)

    predicted = _decoder(h, params['dec'])
    return predicted, tgt


# speedup vs baseline: 1.8513x; 1.8513x over previous
import functools

import jax
import jax.numpy as jnp
from jax import lax
from jax.experimental import pallas as pl
from jax.experimental.pallas import tpu as pltpu
from jax.experimental.pallas import tpu_sc as plsc

H = 128
N_NODES = 10000
N_EDGES = 160000
NW = 32
CHUNK = 40
ACC_ROWS = 10240
NPW = ACC_ROWS // (NW // 2)
EPW = N_EDGES // NW
NCHUNK = EPW // CHUNK

EB = 2000
NB = 2000

_f32 = jnp.float32


def _ln(h, g, beta):
    mu = jnp.mean(h, axis=-1, keepdims=True)
    d = h - mu
    var = jnp.mean(d * d, axis=-1, keepdims=True)
    return d * lax.rsqrt(var + 1e-5) * g + beta


def _dot(a, b):
    return jnp.dot(a, b, preferred_element_type=_f32)


def _node_encoder_body(x_ref, noise_ref, y_ref, w0_ref, b0_ref, w1_ref, b1_ref,
                       w2_ref, b2_ref, g_ref, beta_ref, ws_ref, wr_ref,
                       h_ref, ps_ref, pr_ref, tgt_ref):
    x = x_ref[...]
    noised = x[:, 1:3] + noise_ref[...]
    node_type = x[:, 0:1].astype(jnp.int32)
    cols = lax.broadcasted_iota(jnp.int32, (x.shape[0], 16), 1)
    one_hot = jnp.where((cols - 2) == node_type, 1.0, 0.0).astype(_f32)
    one_hot = jnp.where(cols < 2, 0.0, one_hot)
    one_hot = jnp.where(cols >= 11, 0.0, one_hot)
    feats = jnp.where(cols == 0, noised[:, 0:1], one_hot)
    feats = jnp.where(cols == 1, noised[:, 1:2], feats)
    cnt = feats.shape[0]
    mean = jnp.sum(feats, axis=0, keepdims=True) / cnt
    var = jnp.maximum(jnp.sum(feats * feats, axis=0, keepdims=True) / cnt
                      - mean * mean, 0.0)
    std = jnp.maximum(jnp.sqrt(var), 1e-8)
    na = (feats - mean) / std
    h1 = jnp.maximum(_dot(na, w0_ref[...]) + b0_ref[...], 0.0)
    h2 = jnp.maximum(_dot(h1, w1_ref[...]) + b1_ref[...], 0.0)
    h3 = _dot(h2, w2_ref[...]) + b2_ref[...]
    h = _ln(h3, g_ref[...], beta_ref[...])
    h_ref[...] = h
    ps_ref[...] = _dot(h, ws_ref[...])
    pr_ref[...] = _dot(h, wr_ref[...])
    t = y_ref[...] - noised
    tm = jnp.sum(t, axis=0, keepdims=True) / cnt
    tv = jnp.maximum(jnp.sum(t * t, axis=0, keepdims=True) / cnt - tm * tm, 0.0)
    ts = jnp.maximum(jnp.sqrt(tv), 1e-8)
    tgt_ref[...] = (t - tm) / ts


def _mlp_body(x_ref, w0_ref, b0_ref, w1_ref, b1_ref, w2_ref, b2_ref,
              g_ref, beta_ref, out_ref):
    h1 = jnp.maximum(_dot(x_ref[...], w0_ref[...]) + b0_ref[...], 0.0)
    h2 = jnp.maximum(_dot(h1, w1_ref[...]) + b1_ref[...], 0.0)
    h3 = _dot(h2, w2_ref[...]) + b2_ref[...]
    out_ref[...] = _ln(h3, g_ref[...], beta_ref[...])


def _edge_mlp_body(e_ref, gs_ref, gr_ref, w0_ref, b0_ref, w1_ref, b1_ref,
                   w2_ref, b2_ref, g_ref, beta_ref, enew_ref, eout_ref):
    e = e_ref[...]
    u = _dot(e, w0_ref[...]) + gs_ref[...] + gr_ref[...] + b0_ref[...]
    h1 = jnp.maximum(u, 0.0)
    h2 = jnp.maximum(_dot(h1, w1_ref[...]) + b1_ref[...], 0.0)
    h3 = _dot(h2, w2_ref[...]) + b2_ref[...]
    en = _ln(h3, g_ref[...], beta_ref[...])
    enew_ref[...] = en
    eout_ref[...] = e + en


def _node_mlp_body(h_ref, a0_ref, a1_ref, w0h_ref, w0a_ref,
                   b0_ref, w1_ref, b1_ref, w2_ref, b2_ref, g_ref, beta_ref,
                   ws_ref, wr_ref, hout_ref, ps_ref, pr_ref):
    h = h_ref[...]
    agg = a0_ref[0] + a1_ref[0]
    u = _dot(h, w0h_ref[...]) + _dot(agg, w0a_ref[...]) + b0_ref[...]
    h1 = jnp.maximum(u, 0.0)
    h2 = jnp.maximum(_dot(h1, w1_ref[...]) + b1_ref[...], 0.0)
    h3 = _dot(h2, w2_ref[...]) + b2_ref[...]
    hn = _ln(h3, g_ref[...], beta_ref[...])
    hout = h + hn
    hout_ref[...] = hout
    if ps_ref is not None:
        ps_ref[...] = _dot(hout, ws_ref[...])
        pr_ref[...] = _dot(hout, wr_ref[...])


def _node_mlp_last_body(h_ref, a0_ref, a1_ref, w0h_ref,
                        w0a_ref, b0_ref, w1_ref, b1_ref, w2_ref, b2_ref,
                        g_ref, beta_ref, hout_ref):
    _node_mlp_body(h_ref, a0_ref, a1_ref, w0h_ref, w0a_ref,
                   b0_ref, w1_ref, b1_ref, w2_ref, b2_ref, g_ref, beta_ref,
                   None, None, hout_ref, None, None)


def _decoder_body(h_ref, w0_ref, b0_ref, w1_ref, b1_ref, w2_ref, b2_ref,
                  out_ref):
    h1 = jnp.maximum(_dot(h_ref[...], w0_ref[...]) + b0_ref[...], 0.0)
    h2 = jnp.maximum(_dot(h1, w1_ref[...]) + b1_ref[...], 0.0)
    out_ref[...] = _dot(h2, w2_ref[...]) + b2_ref[...]


def _full(shape):
    return pl.BlockSpec(shape, lambda *_: tuple(0 for _ in shape))


def _node_encoder(x, noise, y, p, ws, wr):
    n = x.shape[0]
    w0p = jnp.zeros((16, H), _f32).at[:p['W0'].shape[0]].set(p['W0'])
    return pl.pallas_call(
        _node_encoder_body,
        out_shape=[jax.ShapeDtypeStruct((n, H), _f32),
                   jax.ShapeDtypeStruct((n, H), _f32),
                   jax.ShapeDtypeStruct((n, H), _f32),
                   jax.ShapeDtypeStruct((n, 2), _f32)],
    )(x, noise, y, w0p, p['b0'], p['W1'], p['b1'], p['W2'], p['b2'],
      p['g'], p['beta'], ws, wr)


def _edge_encoder(edge_attr, p):
    m, k = edge_attr.shape
    grid = (m // EB,)
    w = [_full((k, H)), _full((H,)), _full((H, H)), _full((H,)),
         _full((H, H)), _full((H,)), _full((H,)), _full((H,))]
    return pl.pallas_call(
        _mlp_body,
        grid=grid,
        in_specs=[pl.BlockSpec((EB, k), lambda i: (i, 0))] + w,
        out_specs=pl.BlockSpec((EB, H), lambda i: (i, 0)),
        out_shape=jax.ShapeDtypeStruct((m, H), _f32),
    )(edge_attr, p['W0'], p['b0'], p['W1'], p['b1'], p['W2'], p['b2'],
      p['g'], p['beta'])


def _edge_mlp(e, gs, gr, p):
    m = e.shape[0]
    grid = (m // EB,)
    blk = pl.BlockSpec((EB, H), lambda i: (i, 0))
    w = [_full((H, H)), _full((H,)), _full((H, H)), _full((H,)),
         _full((H, H)), _full((H,)), _full((H,)), _full((H,))]
    w0e = p['W0'][:H]
    return pl.pallas_call(
        _edge_mlp_body,
        grid=grid,
        in_specs=[blk, blk, blk] + w,
        out_specs=[blk, blk],
        out_shape=[jax.ShapeDtypeStruct((m, H), _f32),
                   jax.ShapeDtypeStruct((m, H), _f32)],
    )(e, gs, gr, w0e, p['b0'], p['W1'], p['b1'], p['W2'], p['b2'],
      p['g'], p['beta'])


def _node_mlp(h, agg2, p, ws, wr):
    n = h.shape[0]
    grid = (n // NB,)
    blk = pl.BlockSpec((NB, H), lambda i: (i, 0))
    ablk0 = pl.BlockSpec((1, NB, H), lambda i: (0, i, 0))
    ablk1 = pl.BlockSpec((1, NB, H), lambda i: (1, i, 0))
    w = [_full((H, H)), _full((H, H)), _full((H,)), _full((H, H)),
         _full((H,)), _full((H, H)), _full((H,)), _full((H,)), _full((H,))]
    w0h = p['W0'][:H]
    w0a = p['W0'][H:]
    if ws is None:
        return pl.pallas_call(
            _node_mlp_last_body,
            grid=grid,
            in_specs=[blk, ablk0, ablk1] + w,
            out_specs=blk,
            out_shape=jax.ShapeDtypeStruct((n, H), _f32),
        )(h, agg2, agg2, w0h, w0a, p['b0'], p['W1'], p['b1'],
          p['W2'], p['b2'], p['g'], p['beta'])
    return pl.pallas_call(
        _node_mlp_body,
        grid=grid,
        in_specs=[blk, ablk0, ablk1] + w
        + [_full((H, H)), _full((H, H))],
        out_specs=[blk, blk, blk],
        out_shape=[jax.ShapeDtypeStruct((n, H), _f32),
                   jax.ShapeDtypeStruct((n, H), _f32),
                   jax.ShapeDtypeStruct((n, H), _f32)],
    )(h, agg2, agg2, w0h, w0a, p['b0'], p['W1'], p['b1'],
      p['W2'], p['b2'], p['g'], p['beta'], ws, wr)


def _decoder(h, p):
    n = h.shape[0]
    grid = (n // NB,)
    w = [_full((H, H)), _full((H,)), _full((H, H)), _full((H,)),
         _full((H, 2)), _full((2,))]
    return pl.pallas_call(
        _decoder_body,
        grid=grid,
        in_specs=[pl.BlockSpec((NB, H), lambda i: (i, 0))] + w,
        out_specs=pl.BlockSpec((NB, 2), lambda i: (i, 0)),
        out_shape=jax.ShapeDtypeStruct((n, 2), _f32),
    )(h, p['W0'], p['b0'], p['W1'], p['b1'], p['W2'], p['b2'])


RING = 5


def _sc_gather_body(ps_hbm, pr_hbm, s3_hbm, r3_hbm, gs_hbm, gr_hbm,
                    sidx, ridx, *rest):
    ba = rest[0:RING]
    bb = rest[RING:2 * RING]
    sg = rest[2 * RING:3 * RING]
    sw = rest[3 * RING:4 * RING]
    c = lax.axis_index("c")
    s = lax.axis_index("s")
    wid = s * 2 + c
    base = wid * EPW
    pltpu.sync_copy(s3_hbm.at[wid], sidx)
    pltpu.sync_copy(r3_hbm.at[wid], ridx)

    def fire_g(j, k):
        pltpu.async_copy(ps_hbm.at[sidx.at[j]], ba[k], sg[k])
        pltpu.async_copy(pr_hbm.at[ridx.at[j]], bb[k], sg[k])

    def wait_g(k):
        pltpu.make_async_copy(ps_hbm.at[sidx.at[0]], ba[k], sg[k]).wait()
        pltpu.make_async_copy(pr_hbm.at[ridx.at[0]], bb[k], sg[k]).wait()

    def fire_w(j, k):
        sl = pl.ds(base + j * CHUNK, CHUNK)
        pltpu.async_copy(ba[k], gs_hbm.at[sl], sw[k])
        pltpu.async_copy(bb[k], gr_hbm.at[sl], sw[k])

    def wait_w(k):
        sl = pl.ds(base, CHUNK)
        pltpu.make_async_copy(ba[k], gs_hbm.at[sl], sw[k]).wait()
        pltpu.make_async_copy(bb[k], gr_hbm.at[sl], sw[k]).wait()

    for k in range(RING):
        fire_g(k, k)

    def body(i, carry):
        j0 = RING * i
        for k in range(RING):
            wait_g(k)
            fire_w(j0 + k, k)
        for k in range(RING):
            wait_w(k)
            fire_g(j0 + k + RING, k)
        return carry

    lax.fori_loop(0, NCHUNK // RING - 1, body, 0, unroll=False)
    tail = NCHUNK - RING
    for k in range(RING):
        wait_g(k)
        fire_w(tail + k, k)
    for k in range(RING):
        wait_w(k)


@functools.cache
def _gather_kernel():
    return pl.kernel(
        _sc_gather_body,
        out_type=[jax.ShapeDtypeStruct((N_EDGES, H), _f32),
                  jax.ShapeDtypeStruct((N_EDGES, H), _f32)],
        mesh=plsc.VectorSubcoreMesh(core_axis_name="c", subcore_axis_name="s"),
        scratch_types=(
            [pltpu.VMEM((NCHUNK, CHUNK), jnp.int32)] * 2
            + [pltpu.VMEM((CHUNK, H), _f32)] * (2 * RING)
            + [pltpu.SemaphoreType.DMA] * (2 * RING)
        ),
    )


def _sc_gather(ps, pr, s3, r3):
    return _gather_kernel()(ps, pr, s3, r3)


def _sc_scatter_body(enew_hbm, r3_hbm, out_hbm, ridx, acc, *rest):
    rb = rest[0:RING]
    sl_ = rest[RING:2 * RING]
    sa = rest[2 * RING:3 * RING]
    c = lax.axis_index("c")
    s = lax.axis_index("s")
    wid = s * 2 + c
    base = wid * EPW

    zv = jnp.zeros((16,), _f32)

    def zb(i, carry):
        r = i // 8
        col = lax.rem(i, 8) * 16
        rb[0][r, pl.ds(col, 16)] = zv
        return carry

    lax.fori_loop(0, CHUNK * 8, zb, 0, unroll=False)
    for k in range(NPW // CHUNK):
        pltpu.sync_copy(rb[0], acc.at[pl.ds(s * NPW + k * CHUNK, CHUNK)])
    plsc.subcore_barrier()

    pltpu.sync_copy(r3_hbm.at[wid], ridx)

    def fire_l(j, k):
        pltpu.async_copy(enew_hbm.at[pl.ds(base + j * CHUNK, CHUNK)],
                         rb[k], sl_[k])

    def wait_l(k):
        pltpu.make_async_copy(enew_hbm.at[pl.ds(base, CHUNK)],
                              rb[k], sl_[k]).wait()

    def fire_a(j, k):
        pltpu.async_copy(rb[k], acc.at[ridx.at[j]], sa[k], add=True)

    def wait_a(k):
        pltpu.make_async_copy(rb[k], acc.at[ridx.at[0]], sa[k]).wait()

    for k in range(RING):
        fire_l(k, k)

    def body(i, carry):
        j0 = RING * i
        for k in range(RING):
            wait_l(k)
            fire_a(j0 + k, k)
        for k in range(RING):
            wait_a(k)
            fire_l(j0 + k + RING, k)
        return carry

    lax.fori_loop(0, NCHUNK // RING - 1, body, 0, unroll=False)
    tail = NCHUNK - RING
    for k in range(RING):
        wait_l(k)
        fire_a(tail + k, k)
    for k in range(RING):
        wait_a(k)
    plsc.subcore_barrier()
    for k in range(5):
        sl = pl.ds(s * NPW + k * 128, 128)
        pltpu.sync_copy(acc.at[sl], out_hbm.at[c].at[sl])


@functools.cache
def _scatter_kernel():
    return pl.kernel(
        _sc_scatter_body,
        out_type=jax.ShapeDtypeStruct((2, ACC_ROWS, H), _f32),
        mesh=plsc.VectorSubcoreMesh(core_axis_name="c", subcore_axis_name="s"),
        scratch_types=(
            [pltpu.VMEM((NCHUNK, CHUNK), jnp.int32),
             pltpu.VMEM_SHARED((ACC_ROWS, H), _f32)]
            + [pltpu.VMEM((CHUNK, H), _f32)] * RING
            + [pltpu.SemaphoreType.DMA] * (2 * RING)
        ),
    )


def _sc_scatter(enew, r3):
    return _scatter_kernel()(enew, r3)


def kernel(x, y, edge_attr, velocity_sequence_noise, params, edge_index):
    senders = edge_index[0]
    receivers = edge_index[1]
    s3 = senders.reshape(NW, NCHUNK, CHUNK)
    r3 = receivers.reshape(NW, NCHUNK, CHUNK)

    gn = params['gn']
    w0 = gn[0]['edge']['W0']
    h, ps, pr, tgt = _node_encoder(x, velocity_sequence_noise, y,
                                   params['nb_enc'], w0[H:2 * H], w0[2 * H:])
    e = _edge_encoder(edge_attr, params['eb_enc'])

    for r in range(15):
        gs, gr = _sc_gather(ps, pr, s3, r3)
        e_new, e = _edge_mlp(e, gs, gr, gn[r]['edge'])
        agg2 = _sc_scatter(e_new, r3)
        if r < 14:
            w0n = gn[r + 1]['edge']['W0']
            h, ps, pr = _node_mlp(h, agg2, gn[r]['node'],
                                  w0n[H:2 * H], w0n[2 * H:])
        else:
            h = _node_mlp(h, agg2, gn[r]['node'], None, None)

    predicted = _decoder(h, params['dec'])
    return predicted, tgt
